# Initial kernel scaffold; baseline (speedup 1.0000x reference)
import jax
import jax.numpy as jnp
from jax import lax
from jax.experimental import pallas as pl
from jax.experimental.pallas import tpu as pltpu
from jax.experimental.pallas import tpu_sc as plsc

VOCAB = 1000000
EMBED_DIM = 64
PAD_IDX = 1
B = 1024
S = 512
NC = 2
NS = 16
NW = NC * NS
CHUNK = 128
TPW = (B * S) // NW
CPW = TPW // CHUNK
SEQ_PER_W = TPW // S
PT_ROWS = S + 2


def _embedder_kernel(tok_hbm, postab_hbm, tab_hbm, out_hbm,
                     tok_v, pos_v, postab_v, buf, sem):
    wid = lax.axis_index("s") * NC + lax.axis_index("c")
    row0 = wid * CPW

    pltpu.sync_copy(tok_hbm.at[pl.ds(row0, CPW), :], tok_v)
    pltpu.sync_copy(postab_hbm.at[pl.ds(0, PT_ROWS * EMBED_DIM)], postab_v)

    groups_per_row = CHUNK // 16

    def pos_row(j, carry):
        def pos_group(g, base):
            t = tok_v[j, pl.ds(g * 16, 16)]
            m = t != PAD_IDX
            mi = m.astype(jnp.int32)
            cs = plsc.cumsum(mi)
            pos = jnp.where(m, cs + base, PAD_IDX)
            pos_v[j, pl.ds(g * 16, 16)] = pos
            return base + jnp.sum(mi)
        base0 = jnp.where(j % (S // CHUNK) == 0, PAD_IDX, carry)
        return lax.fori_loop(0, groups_per_row, pos_group, base0)

    lax.fori_loop(0, CPW, pos_row, jnp.int32(PAD_IDX))

    def chunk_body(c, _):
        cp = pltpu.async_copy(tab_hbm.at[tok_v.at[c]], buf, sem)
        cp.wait()

        def add_body(t4, _):
            for u in range(4):
                t = t4 * 4 + u
                p = pos_v[c, t]
                off = p * EMBED_DIM
                for k in range(EMBED_DIM // 16):
                    pv = postab_v[pl.ds(off + k * 16, 16)]
                    plsc.addupdate(buf.at[t, pl.ds(k * 16, 16)], pv)
            return 0

        lax.fori_loop(0, CHUNK // 4, add_body, 0)
        pltpu.sync_copy(buf, out_hbm.at[pl.ds((row0 + c) * CHUNK, CHUNK), :])
        return 0

    lax.fori_loop(0, CPW, chunk_body, 0)


def kernel(src_tokens, src_lengths, embed_tokens, embed_positions):
    del src_lengths
    tok2d = src_tokens.reshape(B * S // CHUNK, CHUNK)
    postab_flat = embed_positions.reshape(-1)

    mesh = plsc.VectorSubcoreMesh(
        core_axis_name="c", subcore_axis_name="s",
        num_cores=NC, num_subcores=NS)

    run = pl.kernel(
        _embedder_kernel,
        out_type=jax.ShapeDtypeStruct((B * S, EMBED_DIM), jnp.float32),
        mesh=mesh,
        scratch_types=[
            pltpu.VMEM((CPW, CHUNK), jnp.int32),
            pltpu.VMEM((CPW, CHUNK), jnp.int32),
            pltpu.VMEM((PT_ROWS * EMBED_DIM,), jnp.float32),
            pltpu.VMEM((CHUNK, EMBED_DIM), jnp.float32),
            pltpu.SemaphoreType.DMA,
        ],
    )
    x = run(tok2d, postab_flat, embed_tokens).reshape(B, S, EMBED_DIM)
    return (x, x)


# speedup vs baseline: 1.6598x; 1.6598x over previous
import jax
import jax.numpy as jnp
from jax import lax
from jax.experimental import pallas as pl
from jax.experimental.pallas import tpu as pltpu
from jax.experimental.pallas import tpu_sc as plsc

VOCAB = 1000000
EMBED_DIM = 64
PAD_IDX = 1
B = 1024
S = 512
NC = 2
NS = 16
NW = NC * NS
CHUNK = 128
TPW = (B * S) // NW
CPW = TPW // CHUNK
PT_ROWS = S + 2


def _embedder_kernel(tok_hbm, postab_hbm, tab_hbm, out_hbm,
                     tok_v, pos_v, idx_v, postab_v, buf, ob, sem):
    wid = lax.axis_index("s") * NC + lax.axis_index("c")
    row0 = wid * CPW

    pltpu.sync_copy(tok_hbm.at[pl.ds(row0, CPW), :], tok_v)
    pltpu.sync_copy(postab_hbm.at[pl.ds(0, PT_ROWS * EMBED_DIM)], postab_v)

    groups_per_row = CHUNK // 16

    def pos_row(j, carry):
        def pos_group(g, base):
            t = tok_v[j, pl.ds(g * 16, 16)]
            idx_v[j, pl.ds(g * 16, 16)] = t >> 1
            mi = jnp.minimum(jnp.abs(t - PAD_IDX), 1)
            cs = plsc.cumsum(mi)
            pos_v[j, pl.ds(g * 16, 16)] = mi * (cs + base - PAD_IDX) + PAD_IDX
            return base + jnp.sum(mi)
        base0 = jnp.where(j % (S // CHUNK) == 0, jnp.int32(PAD_IDX), carry)
        return lax.fori_loop(0, groups_per_row, pos_group, base0)

    lax.fori_loop(0, CPW, pos_row, jnp.int32(PAD_IDX))

    def chunk_body(c, _):
        cp = pltpu.async_copy(tab_hbm.at[idx_v.at[c]], buf, sem)
        cp.wait()

        def add_body(g, _):
            p16 = pos_v[c, pl.ds(g * 16, 16)] * EMBED_DIM
            h16 = (tok_v[c, pl.ds(g * 16, 16)] & 1) * EMBED_DIM
            for u in range(16):
                t = g * 16 + u
                poff = p16[u]
                hoff = h16[u]
                for k in range(EMBED_DIM // 16):
                    tv = buf[t, pl.ds(hoff + k * 16, 16)]
                    pv = postab_v[pl.ds(poff + k * 16, 16)]
                    ob[t, pl.ds(k * 16, 16)] = tv + pv
            return 0

        lax.fori_loop(0, CHUNK // 16, add_body, 0)
        pltpu.sync_copy(ob, out_hbm.at[pl.ds((row0 + c) * CHUNK, CHUNK), :])
        return 0

    lax.fori_loop(0, CPW, chunk_body, 0)


def kernel(src_tokens, src_lengths, embed_tokens, embed_positions):
    del src_lengths
    tok2d = src_tokens.reshape(B * S // CHUNK, CHUNK)
    postab_flat = embed_positions.reshape(-1)
    tab2 = embed_tokens.reshape(VOCAB // 2, 2 * EMBED_DIM)

    mesh = plsc.VectorSubcoreMesh(
        core_axis_name="c", subcore_axis_name="s",
        num_cores=NC, num_subcores=NS)

    run = pl.kernel(
        _embedder_kernel,
        out_type=jax.ShapeDtypeStruct((B * S, EMBED_DIM), jnp.float32),
        mesh=mesh,
        compiler_params=pltpu.CompilerParams(needs_layout_passes=False),
        scratch_types=[
            pltpu.VMEM((CPW, CHUNK), jnp.int32),
            pltpu.VMEM((CPW, CHUNK), jnp.int32),
            pltpu.VMEM((CPW, CHUNK), jnp.int32),
            pltpu.VMEM((PT_ROWS * EMBED_DIM,), jnp.float32),
            pltpu.VMEM((CHUNK, 2 * EMBED_DIM), jnp.float32),
            pltpu.VMEM((CHUNK, EMBED_DIM), jnp.float32),
            pltpu.SemaphoreType.DMA,
        ],
    )
    x = run(tok2d, postab_flat, tab2).reshape(B, S, EMBED_DIM)
    return (x, x)


# speedup vs baseline: 2.0457x; 1.2325x over previous
import jax
import jax.numpy as jnp
from jax import lax
from jax.experimental import pallas as pl
from jax.experimental.pallas import tpu as pltpu
from jax.experimental.pallas import tpu_sc as plsc

VOCAB = 1000000
EMBED_DIM = 64
PAD_IDX = 1
B = 1024
S = 512
NC = 2
NS = 16
NW = NC * NS
CHUNK = 128
TPW = (B * S) // NW
CPW = TPW // CHUNK
GPC = CHUNK // 16
CPS = S // CHUNK
PT_ROWS = S + 2
NBUF = 3
NSTEP = (CPW + NBUF - 1) // NBUF


def _embedder_kernel(tok_hbm, postab_hbm, tab_hbm, out_hbm,
                     tokst, idx3, pk3, postab_v, buf3, ob2,
                     tsem, gsem, osem):
    wid = lax.axis_index("s") * NC + lax.axis_index("c")
    row0 = wid * CPW

    pltpu.sync_copy(postab_hbm.at[pl.ds(0, PT_ROWS * EMBED_DIM)], postab_v)

    def tok_start(c, s):
        pltpu.async_copy(
            tok_hbm.at[pl.ds(row0 + c, 1), :], tokst.at[s], tsem.at[s])

    def tok_wait(s):
        pltpu.make_async_copy(
            tok_hbm.at[pl.ds(row0, 1), :], tokst.at[s], tsem.at[s]).wait()

    def idx_stage(s, c, base_in):
        def grp(g, base):
            t = tokst[s, 0, pl.ds(g * 16, 16)]
            idx3[s, pl.ds(g * 16, 16)] = t >> 1
            mi = jnp.minimum(jnp.abs(t - PAD_IDX), 1)
            cs = plsc.cumsum(mi)
            pos = mi * (cs + base - PAD_IDX) + PAD_IDX
            hoff = (t & 1) << 6
            pk3[s, pl.ds(g * 16, 16)] = (hoff << 16) | (pos * EMBED_DIM)
            return base + jnp.sum(mi)
        base0 = jnp.where(c % CPS == 0, jnp.int32(PAD_IDX), base_in)
        return lax.fori_loop(0, GPC, grp, base0)

    def gat_start(s):
        pltpu.async_copy(tab_hbm.at[idx3.at[s]], buf3.at[s], gsem.at[s])

    def gat_wait(s):
        pltpu.make_async_copy(
            tab_hbm.at[idx3.at[s]], buf3.at[s], gsem.at[s]).wait()

    def out_start(c, s):
        pltpu.async_copy(
            ob2.at[s], out_hbm.at[pl.ds((row0 + c) * CHUNK, CHUNK), :],
            osem.at[s])

    def out_wait(s):
        pltpu.make_async_copy(
            ob2.at[s], out_hbm.at[pl.ds(0, CHUNK), :], osem.at[s]).wait()

    def add_stage(s, w_):
        def grp(g, _):
            w16 = pk3[s, pl.ds(g * 16, 16)]
            for u in range(16):
                t = g * 16 + u
                w = w16[u]
                poff = w & 0xFFFF
                hoff = w >> 16
                for k in range(EMBED_DIM // 16):
                    tv = buf3[s, t, pl.ds(hoff + k * 16, 16)]
                    pv = postab_v[pl.ds(poff + k * 16, 16)]
                    ob2[w_, t, pl.ds(k * 16, 16)] = tv + pv
            return 0
        lax.fori_loop(0, GPC, grp, 0)

    tok_start(0, 0)
    tok_start(1, 1)
    tok_start(2, 2)
    tok_wait(0)
    base = idx_stage(0, jnp.int32(0), jnp.int32(PAD_IDX))
    tok_wait(1)
    base = idx_stage(1, jnp.int32(1), base)
    gat_start(0)

    def step(q, base_in):
        base = base_in
        for b in range(NBUF):
            c = q * NBUF + b

            @pl.when(c + 3 < CPW)
            def _():
                tok_start(c + 3, b)

            @pl.when(c + 2 < CPW)
            def _():
                tok_wait((b + 2) % NBUF)
            base = jnp.where(
                c + 2 < CPW,
                idx_stage((b + 2) % NBUF, c + 2, base), base)

            @pl.when(c + 1 < CPW)
            def _():
                gat_start((b + 1) % NBUF)

            ow = (c + q) * 0 + (q + b) % 2

            @pl.when(jnp.logical_and(c < CPW, c >= 2))
            def _():
                out_wait(ow)

            @pl.when(c < CPW)
            def _():
                gat_wait(b)

            @pl.when(c < CPW)
            def _():
                add_stage(b, ow)
                out_start(c, ow)
        return base

    lax.fori_loop(0, NSTEP, step, base)

    out_wait(0)
    out_wait(1)


def kernel(src_tokens, src_lengths, embed_tokens, embed_positions):
    del src_lengths
    tok2d = src_tokens.reshape(B * S // CHUNK, CHUNK)
    postab_flat = embed_positions.reshape(-1)
    tab2 = embed_tokens.reshape(VOCAB // 2, 2 * EMBED_DIM)

    mesh = plsc.VectorSubcoreMesh(
        core_axis_name="c", subcore_axis_name="s",
        num_cores=NC, num_subcores=NS)

    run = pl.kernel(
        _embedder_kernel,
        out_type=jax.ShapeDtypeStruct((B * S, EMBED_DIM), jnp.float32),
        mesh=mesh,
        compiler_params=pltpu.CompilerParams(needs_layout_passes=False),
        scratch_types=[
            pltpu.VMEM((NBUF, 1, CHUNK), jnp.int32),
            pltpu.VMEM((NBUF, CHUNK), jnp.int32),
            pltpu.VMEM((NBUF, CHUNK), jnp.int32),
            pltpu.VMEM((PT_ROWS * EMBED_DIM,), jnp.float32),
            pltpu.VMEM((NBUF, CHUNK, 2 * EMBED_DIM), jnp.float32),
            pltpu.VMEM((2, CHUNK, EMBED_DIM), jnp.float32),
            pltpu.SemaphoreType.DMA((NBUF,)),
            pltpu.SemaphoreType.DMA((NBUF,)),
            pltpu.SemaphoreType.DMA((2,)),
        ],
    )
    x = run(tok2d, postab_flat, tab2).reshape(B, S, EMBED_DIM)
    return (x, x)
